# Initial kernel scaffold; baseline (speedup 1.0000x reference)
import jax
import jax.numpy as jnp
from jax.experimental import pallas as pl


def kernel(xyz1, xyz2, points1, points2, W, gamma, beta):
    raise NotImplementedError("write your pallas kernel here")


# speedup vs baseline: 10.6808x; 10.6808x over previous
import functools

import jax
import jax.numpy as jnp
from jax.experimental import pallas as pl


def _kernel_a(xyz1_ref, xyz2_ref, p1_ref, p2_ref, w_ref,
              pre_ref, part_ref):
    d1 = p1_ref.shape[1]

    q = xyz1_ref[0]
    k = xyz2_ref[0]
    qb = q.astype(jnp.bfloat16).astype(jnp.float32)
    kb = k.astype(jnp.bfloat16).astype(jnp.float32)
    cross = jax.lax.dot_general(
        qb, kb, (((1,), (1,)), ((), ())),
        precision=jax.lax.Precision.HIGHEST,
        preferred_element_type=jnp.float32)
    q2 = jnp.sum(q * q, axis=1)[:, None]
    k2 = jnp.sum(k * k, axis=1)[None, :]
    d2 = q2 + k2 - 2.0 * cross

    big = jnp.float32(jnp.inf)
    m1 = jnp.min(d2, axis=1, keepdims=True)
    d2b = jnp.where(d2 == m1, big, d2)
    m2 = jnp.min(d2b, axis=1, keepdims=True)
    d2c = jnp.where(d2b == m2, big, d2b)
    m3 = jnp.min(d2c, axis=1, keepdims=True)

    r1 = 1.0 / (jnp.sqrt(jnp.maximum(m1, 1e-12)) + 1e-8)
    r2 = 1.0 / (jnp.sqrt(jnp.maximum(m2, 1e-12)) + 1e-8)
    r3 = 1.0 / (jnp.sqrt(jnp.maximum(m3, 1e-12)) + 1e-8)
    norm = r1 + r2 + r3

    zero = jnp.float32(0.0)
    wmat = (jnp.where(d2 == m1, r1 / norm, zero)
            + jnp.where(d2b == m2, r2 / norm, zero)
            + jnp.where(d2c == m3, r3 / norm, zero))

    p2 = p2_ref[0]
    interp = jax.lax.dot_general(
        wmat, p2, (((1,), (1,)), ((), ())),
        precision=jax.lax.Precision.HIGHEST,
        preferred_element_type=jnp.float32)

    p1 = p1_ref[0]
    w_all = w_ref[...]
    pre = jax.lax.dot_general(
        w_all[:, :d1], p1, (((1,), (0,)), ((), ())),
        precision=jax.lax.Precision.HIGHEST,
        preferred_element_type=jnp.float32)
    pre = pre + jax.lax.dot_general(
        w_all[:, d1:], interp, (((1,), (1,)), ((), ())),
        precision=jax.lax.Precision.HIGHEST,
        preferred_element_type=jnp.float32)

    pre_ref[0] = pre
    part_ref[0, :, 0:1] = jnp.sum(pre, axis=1, keepdims=True)
    part_ref[0, :, 1:2] = jnp.sum(pre * pre, axis=1, keepdims=True)


def _kernel_b(pre_ref, part_ref, gamma_ref, beta_ref, out_ref, *, inv_count):
    totals = jnp.sum(part_ref[...], axis=0)
    mean = totals[:, 0:1] * inv_count
    ex2 = totals[:, 1:2] * inv_count
    var = ex2 - mean * mean
    scale = gamma_ref[...] / jnp.sqrt(var + 1e-5)
    shift = beta_ref[...] - mean * scale
    x = pre_ref[0]
    out_ref[0] = jnp.maximum(x * scale + shift, 0.0)


def kernel(xyz1, xyz2, points1, points2, W, gamma, beta):
    B, N, _ = xyz1.shape
    S = xyz2.shape[1]
    D1 = points1.shape[1]
    D2 = points2.shape[1]
    O, C = W.shape

    TN = min(512, N)
    NT = N // TN

    pre, part = pl.pallas_call(
        _kernel_a,
        grid=(B, NT),
        in_specs=[
            pl.BlockSpec((1, TN, 3), lambda b, n: (b, n, 0)),
            pl.BlockSpec((1, S, 3), lambda b, n: (b, 0, 0)),
            pl.BlockSpec((1, D1, TN), lambda b, n: (b, 0, n)),
            pl.BlockSpec((1, D2, S), lambda b, n: (b, 0, 0)),
            pl.BlockSpec((O, C), lambda b, n: (0, 0)),
        ],
        out_specs=[
            pl.BlockSpec((1, O, TN), lambda b, n: (b, 0, n)),
            pl.BlockSpec((1, O, 8), lambda b, n: (b * NT + n, 0, 0)),
        ],
        out_shape=[
            jax.ShapeDtypeStruct((B, O, N), jnp.float32),
            jax.ShapeDtypeStruct((B * NT, O, 8), jnp.float32),
        ],
    )(xyz1, xyz2, points1, points2, W)

    out = pl.pallas_call(
        functools.partial(_kernel_b, inv_count=1.0 / (B * N)),
        grid=(B, NT),
        in_specs=[
            pl.BlockSpec((1, O, TN), lambda b, n: (b, 0, n)),
            pl.BlockSpec((B * NT, O, 8), lambda b, n: (0, 0, 0)),
            pl.BlockSpec((O, 1), lambda b, n: (0, 0)),
            pl.BlockSpec((O, 1), lambda b, n: (0, 0)),
        ],
        out_specs=pl.BlockSpec((1, O, TN), lambda b, n: (b, 0, n)),
        out_shape=jax.ShapeDtypeStruct((B, O, N), jnp.float32),
    )(pre, part, gamma.reshape(O, 1), beta.reshape(O, 1))

    return out


# speedup vs baseline: 20.8585x; 1.9529x over previous
import functools

import jax
import jax.numpy as jnp
from jax.experimental import pallas as pl


def _kernel_a(xyz1_ref, xyz2_ref, p1_ref, p2_ref, w_ref,
              pre_ref, part_ref):
    d1 = p1_ref.shape[1]

    q = xyz1_ref[0]
    k = xyz2_ref[0]
    qb = q.astype(jnp.bfloat16).astype(jnp.float32)
    kb = k.astype(jnp.bfloat16).astype(jnp.float32)
    cross = jax.lax.dot_general(
        qb, kb, (((1,), (1,)), ((), ())),
        precision=jax.lax.Precision.HIGHEST,
        preferred_element_type=jnp.float32)
    q2 = jnp.sum(q * q, axis=1)[:, None]
    k2 = jnp.sum(k * k, axis=1)[None, :]
    d2 = q2 + k2 - 2.0 * cross

    big = jnp.float32(jnp.inf)
    m1 = jnp.min(d2, axis=1, keepdims=True)
    d2b = jnp.where(d2 == m1, big, d2)
    m2 = jnp.min(d2b, axis=1, keepdims=True)
    d2c = jnp.where(d2b == m2, big, d2b)
    m3 = jnp.min(d2c, axis=1, keepdims=True)

    r1 = 1.0 / (jnp.sqrt(jnp.maximum(m1, 1e-12)) + 1e-8)
    r2 = 1.0 / (jnp.sqrt(jnp.maximum(m2, 1e-12)) + 1e-8)
    r3 = 1.0 / (jnp.sqrt(jnp.maximum(m3, 1e-12)) + 1e-8)
    inv_norm = 1.0 / (r1 + r2 + r3)

    zero = jnp.float32(0.0)
    wmat = jnp.where(d2 == m1, r1 * inv_norm,
                     jnp.where(d2b == m2, r2 * inv_norm,
                               jnp.where(d2c == m3, r3 * inv_norm, zero)))

    p2 = p2_ref[0]
    interp = jax.lax.dot_general(
        wmat.astype(jnp.bfloat16), p2.astype(jnp.bfloat16),
        (((1,), (1,)), ((), ())),
        preferred_element_type=jnp.float32)

    p1 = p1_ref[0]
    w_all = w_ref[...].astype(jnp.bfloat16)
    pre = jax.lax.dot_general(
        w_all[:, :d1], p1.astype(jnp.bfloat16), (((1,), (0,)), ((), ())),
        preferred_element_type=jnp.float32)
    pre = pre + jax.lax.dot_general(
        w_all[:, d1:], interp.astype(jnp.bfloat16), (((1,), (1,)), ((), ())),
        preferred_element_type=jnp.float32)

    pre_ref[0] = pre
    part_ref[0, :, 0:1] = jnp.sum(pre, axis=1, keepdims=True)
    part_ref[0, :, 1:2] = jnp.sum(pre * pre, axis=1, keepdims=True)


def _kernel_c(part_ref, gamma_ref, beta_ref, ss_ref, *, inv_count):
    totals = jnp.sum(part_ref[...], axis=0)
    mean = totals[:, 0:1] * inv_count
    ex2 = totals[:, 1:2] * inv_count
    var = ex2 - mean * mean
    scale = gamma_ref[...] / jnp.sqrt(var + 1e-5)
    shift = beta_ref[...] - mean * scale
    ss_ref[:, 0:1] = scale
    ss_ref[:, 1:2] = shift


def _kernel_b(pre_ref, ss_ref, out_ref):
    scale = ss_ref[:, 0:1]
    shift = ss_ref[:, 1:2]
    x = pre_ref[0]
    out_ref[0] = jnp.maximum(x * scale + shift, 0.0)


def kernel(xyz1, xyz2, points1, points2, W, gamma, beta):
    B, N, _ = xyz1.shape
    S = xyz2.shape[1]
    D1 = points1.shape[1]
    D2 = points2.shape[1]
    O, C = W.shape

    TN = min(512, N)
    NT = N // TN

    pre, part = pl.pallas_call(
        _kernel_a,
        grid=(B, NT),
        in_specs=[
            pl.BlockSpec((1, TN, 3), lambda b, n: (b, n, 0)),
            pl.BlockSpec((1, S, 3), lambda b, n: (b, 0, 0)),
            pl.BlockSpec((1, D1, TN), lambda b, n: (b, 0, n)),
            pl.BlockSpec((1, D2, S), lambda b, n: (b, 0, 0)),
            pl.BlockSpec((O, C), lambda b, n: (0, 0)),
        ],
        out_specs=[
            pl.BlockSpec((1, O, TN), lambda b, n: (b, 0, n)),
            pl.BlockSpec((1, O, 8), lambda b, n: (b * NT + n, 0, 0)),
        ],
        out_shape=[
            jax.ShapeDtypeStruct((B, O, N), jnp.float32),
            jax.ShapeDtypeStruct((B * NT, O, 8), jnp.float32),
        ],
    )(xyz1, xyz2, points1, points2, W)

    ss = pl.pallas_call(
        functools.partial(_kernel_c, inv_count=1.0 / (B * N)),
        grid=(1,),
        in_specs=[
            pl.BlockSpec((B * NT, O, 8), lambda i: (0, 0, 0)),
            pl.BlockSpec((O, 1), lambda i: (0, 0)),
            pl.BlockSpec((O, 1), lambda i: (0, 0)),
        ],
        out_specs=pl.BlockSpec((O, 8), lambda i: (0, 0)),
        out_shape=jax.ShapeDtypeStruct((O, 8), jnp.float32),
    )(part, gamma.reshape(O, 1), beta.reshape(O, 1))

    out = pl.pallas_call(
        _kernel_b,
        grid=(B, NT),
        in_specs=[
            pl.BlockSpec((1, O, TN), lambda b, n: (b, 0, n)),
            pl.BlockSpec((O, 8), lambda b, n: (0, 0)),
        ],
        out_specs=pl.BlockSpec((1, O, TN), lambda b, n: (b, 0, n)),
        out_shape=jax.ShapeDtypeStruct((B, O, N), jnp.float32),
    )(pre, ss)

    return out


# speedup vs baseline: 21.4822x; 1.0299x over previous
import functools

import jax
import jax.numpy as jnp
from jax.experimental import pallas as pl
from jax.experimental.pallas import tpu as pltpu


def _kernel_a(xyz1_ref, xyz2_ref, p1_ref, p2_ref, w_ref,
              pre_ref, part_ref, z_ref):
    d1 = p1_ref.shape[1]
    nt = pl.program_id(1)

    @pl.when(nt == 0)
    def _compute_z():
        w2 = w_ref[...][:, d1:].astype(jnp.bfloat16)
        p2b = p2_ref[0].astype(jnp.bfloat16)
        z_ref[...] = jax.lax.dot_general(
            w2, p2b, (((1,), (0,)), ((), ())),
            preferred_element_type=jnp.float32).astype(jnp.bfloat16)

    q = xyz1_ref[0]
    k = xyz2_ref[0]
    qb = q.astype(jnp.bfloat16).astype(jnp.float32)
    kb = k.astype(jnp.bfloat16).astype(jnp.float32)
    cross = jax.lax.dot_general(
        qb, kb, (((1,), (1,)), ((), ())),
        precision=jax.lax.Precision.HIGHEST,
        preferred_element_type=jnp.float32)
    q2 = jnp.sum(q * q, axis=1)[:, None]
    k2 = jnp.sum(k * k, axis=1)[None, :]
    d2 = q2 + k2 - 2.0 * cross

    big = jnp.float32(jnp.inf)
    m1 = jnp.min(d2, axis=1, keepdims=True)
    d2b = jnp.where(d2 == m1, big, d2)
    m2 = jnp.min(d2b, axis=1, keepdims=True)
    d2c = jnp.where(d2b == m2, big, d2b)
    m3 = jnp.min(d2c, axis=1, keepdims=True)

    r1 = 1.0 / (jnp.sqrt(jnp.maximum(m1, 1e-12)) + 1e-8)
    r2 = 1.0 / (jnp.sqrt(jnp.maximum(m2, 1e-12)) + 1e-8)
    r3 = 1.0 / (jnp.sqrt(jnp.maximum(m3, 1e-12)) + 1e-8)
    inv_norm = 1.0 / (r1 + r2 + r3)

    zero = jnp.float32(0.0)
    wmat = jnp.where(d2 == m1, r1 * inv_norm,
                     jnp.where(d2b == m2, r2 * inv_norm,
                               jnp.where(d2c == m3, r3 * inv_norm,
                                         zero))).astype(jnp.bfloat16)

    p1 = p1_ref[0]
    w1m = w_ref[...][:, :d1].astype(jnp.bfloat16)
    pre = jax.lax.dot_general(
        w1m, p1.astype(jnp.bfloat16), (((1,), (0,)), ((), ())),
        preferred_element_type=jnp.float32)
    pre = pre + jax.lax.dot_general(
        z_ref[...], wmat, (((1,), (1,)), ((), ())),
        preferred_element_type=jnp.float32)

    pre_ref[0] = pre
    part_ref[0, :, 0:1] = jnp.sum(pre, axis=1, keepdims=True)
    part_ref[0, :, 1:2] = jnp.sum(pre * pre, axis=1, keepdims=True)


def _kernel_c(part_ref, gamma_ref, beta_ref, ss_ref, *, inv_count):
    totals = jnp.sum(part_ref[...], axis=0)
    mean = totals[:, 0:1] * inv_count
    ex2 = totals[:, 1:2] * inv_count
    var = ex2 - mean * mean
    scale = gamma_ref[...] / jnp.sqrt(var + 1e-5)
    shift = beta_ref[...] - mean * scale
    ss_ref[:, 0:1] = scale
    ss_ref[:, 1:2] = shift


def _kernel_b(pre_ref, ss_ref, out_ref):
    scale = ss_ref[:, 0:1]
    shift = ss_ref[:, 1:2]
    x = pre_ref[0]
    out_ref[0] = jnp.maximum(x * scale + shift, 0.0)


def kernel(xyz1, xyz2, points1, points2, W, gamma, beta):
    B, N, _ = xyz1.shape
    S = xyz2.shape[1]
    D1 = points1.shape[1]
    D2 = points2.shape[1]
    O, C = W.shape

    TN = min(512, N)
    NT = N // TN

    pre, part = pl.pallas_call(
        _kernel_a,
        grid=(B, NT),
        in_specs=[
            pl.BlockSpec((1, TN, 3), lambda b, n: (b, n, 0)),
            pl.BlockSpec((1, S, 3), lambda b, n: (b, 0, 0)),
            pl.BlockSpec((1, D1, TN), lambda b, n: (b, 0, n)),
            pl.BlockSpec((1, D2, S), lambda b, n: (b, 0, 0)),
            pl.BlockSpec((O, C), lambda b, n: (0, 0)),
        ],
        out_specs=[
            pl.BlockSpec((1, O, TN), lambda b, n: (b, 0, n)),
            pl.BlockSpec((1, O, 8), lambda b, n: (b * NT + n, 0, 0)),
        ],
        out_shape=[
            jax.ShapeDtypeStruct((B, O, N), jnp.float32),
            jax.ShapeDtypeStruct((B * NT, O, 8), jnp.float32),
        ],
        scratch_shapes=[pltpu.VMEM((O, S), jnp.bfloat16)],
    )(xyz1, xyz2, points1, points2, W)

    ss = pl.pallas_call(
        functools.partial(_kernel_c, inv_count=1.0 / (B * N)),
        grid=(1,),
        in_specs=[
            pl.BlockSpec((B * NT, O, 8), lambda i: (0, 0, 0)),
            pl.BlockSpec((O, 1), lambda i: (0, 0)),
            pl.BlockSpec((O, 1), lambda i: (0, 0)),
        ],
        out_specs=pl.BlockSpec((O, 8), lambda i: (0, 0)),
        out_shape=jax.ShapeDtypeStruct((O, 8), jnp.float32),
    )(part, gamma.reshape(O, 1), beta.reshape(O, 1))

    out = pl.pallas_call(
        _kernel_b,
        grid=(B, NT),
        in_specs=[
            pl.BlockSpec((1, O, TN), lambda b, n: (b, 0, n)),
            pl.BlockSpec((O, 8), lambda b, n: (0, 0)),
        ],
        out_specs=pl.BlockSpec((1, O, TN), lambda b, n: (b, 0, n)),
        out_shape=jax.ShapeDtypeStruct((B, O, N), jnp.float32),
    )(pre, ss)

    return out


# speedup vs baseline: 33.6190x; 1.5650x over previous
import functools

import jax
import jax.numpy as jnp
from jax.experimental import pallas as pl
from jax.experimental.pallas import tpu as pltpu


def _kernel_a(xyz1_ref, xyz2_ref, p1_ref, p2_ref, w_ref,
              pre_ref, part_ref, z_ref):
    d1 = p1_ref.shape[1]
    nt = pl.program_id(1)

    @pl.when(nt == 0)
    def _compute_z():
        w2 = w_ref[...][:, d1:].astype(jnp.bfloat16)
        p2b = p2_ref[0].astype(jnp.bfloat16)
        z_ref[...] = jax.lax.dot_general(
            w2, p2b, (((1,), (0,)), ((), ())),
            preferred_element_type=jnp.float32).astype(jnp.bfloat16)

    q = xyz1_ref[0]
    k = xyz2_ref[0]
    cross = jax.lax.dot_general(
        q.astype(jnp.bfloat16), k.astype(jnp.bfloat16), (((1,), (1,)), ((), ())),
        preferred_element_type=jnp.float32)
    q2 = jnp.sum(q * q, axis=1)[:, None]
    k2 = jnp.sum(k * k, axis=1)[None, :]
    d2 = q2 + k2 - 2.0 * cross

    big = jnp.float32(jnp.inf)
    m1 = jnp.min(d2, axis=1, keepdims=True)
    eq1 = d2 == m1
    d2b = jnp.where(eq1, big, d2)
    m2 = jnp.min(d2b, axis=1, keepdims=True)
    eq2 = d2b == m2
    d2c = jnp.where(eq2, big, d2b)
    m3 = jnp.min(d2c, axis=1, keepdims=True)
    eq3 = d2c == m3

    r1 = 1.0 / (jnp.sqrt(jnp.maximum(m1, 1e-12)) + 1e-8)
    r2 = 1.0 / (jnp.sqrt(jnp.maximum(m2, 1e-12)) + 1e-8)
    r3 = 1.0 / (jnp.sqrt(jnp.maximum(m3, 1e-12)) + 1e-8)
    inv_norm = 1.0 / (r1 + r2 + r3)

    zero = jnp.float32(0.0)
    wmat = jnp.where(eq1, r1 * inv_norm,
                     jnp.where(eq2, r2 * inv_norm,
                               jnp.where(eq3, r3 * inv_norm,
                                         zero))).astype(jnp.bfloat16)

    p1 = p1_ref[0]
    w1m = w_ref[...][:, :d1].astype(jnp.bfloat16)
    pre = jax.lax.dot_general(
        w1m, p1.astype(jnp.bfloat16), (((1,), (0,)), ((), ())),
        preferred_element_type=jnp.float32)
    pre = pre + jax.lax.dot_general(
        z_ref[...], wmat, (((1,), (1,)), ((), ())),
        preferred_element_type=jnp.float32)

    pre_ref[0] = pre
    part_ref[0, :, 0:1] = jnp.sum(pre, axis=1, keepdims=True)
    part_ref[0, :, 1:2] = jnp.sum(pre * pre, axis=1, keepdims=True)


def _kernel_c(part_ref, gamma_ref, beta_ref, ss_ref, *, inv_count):
    totals = jnp.sum(part_ref[...], axis=0)
    mean = totals[:, 0:1] * inv_count
    ex2 = totals[:, 1:2] * inv_count
    var = ex2 - mean * mean
    scale = gamma_ref[...] / jnp.sqrt(var + 1e-5)
    shift = beta_ref[...] - mean * scale
    ss_ref[:, 0:1] = scale
    ss_ref[:, 1:2] = shift


def _kernel_b(pre_ref, ss_ref, out_ref):
    scale = ss_ref[:, 0:1]
    shift = ss_ref[:, 1:2]
    x = pre_ref[0]
    out_ref[0] = jnp.maximum(x * scale + shift, 0.0)


def kernel(xyz1, xyz2, points1, points2, W, gamma, beta):
    B, N, _ = xyz1.shape
    S = xyz2.shape[1]
    D1 = points1.shape[1]
    D2 = points2.shape[1]
    O, C = W.shape

    TN = min(1024, N)
    NT = N // TN

    pre, part = pl.pallas_call(
        _kernel_a,
        grid=(B, NT),
        in_specs=[
            pl.BlockSpec((1, TN, 3), lambda b, n: (b, n, 0)),
            pl.BlockSpec((1, S, 3), lambda b, n: (b, 0, 0)),
            pl.BlockSpec((1, D1, TN), lambda b, n: (b, 0, n)),
            pl.BlockSpec((1, D2, S), lambda b, n: (b, 0, 0)),
            pl.BlockSpec((O, C), lambda b, n: (0, 0)),
        ],
        out_specs=[
            pl.BlockSpec((1, O, TN), lambda b, n: (b, 0, n)),
            pl.BlockSpec((1, O, 8), lambda b, n: (b * NT + n, 0, 0)),
        ],
        out_shape=[
            jax.ShapeDtypeStruct((B, O, N), jnp.float32),
            jax.ShapeDtypeStruct((B * NT, O, 8), jnp.float32),
        ],
        scratch_shapes=[pltpu.VMEM((O, S), jnp.bfloat16)],
    )(xyz1, xyz2, points1, points2, W)

    ss = pl.pallas_call(
        functools.partial(_kernel_c, inv_count=1.0 / (B * N)),
        grid=(1,),
        in_specs=[
            pl.BlockSpec((B * NT, O, 8), lambda i: (0, 0, 0)),
            pl.BlockSpec((O, 1), lambda i: (0, 0)),
            pl.BlockSpec((O, 1), lambda i: (0, 0)),
        ],
        out_specs=pl.BlockSpec((O, 8), lambda i: (0, 0)),
        out_shape=jax.ShapeDtypeStruct((O, 8), jnp.float32),
    )(part, gamma.reshape(O, 1), beta.reshape(O, 1))

    out = pl.pallas_call(
        _kernel_b,
        grid=(B, NT),
        in_specs=[
            pl.BlockSpec((1, O, TN), lambda b, n: (b, 0, n)),
            pl.BlockSpec((O, 8), lambda b, n: (0, 0)),
        ],
        out_specs=pl.BlockSpec((1, O, TN), lambda b, n: (b, 0, n)),
        out_shape=jax.ShapeDtypeStruct((B, O, N), jnp.float32),
    )(pre, ss)

    return out
